# baseline (speedup 1.0000x reference)
import functools

import jax
import jax.numpy as jnp
from jax import lax
from jax.experimental import pallas as pl
from jax.experimental.pallas import tpu as pltpu
from jax.experimental.pallas import tpu_sc as plsc

B = 32
N = 8192
NG = 256
GS = 32


def _fps_body(pc_ref, far0_ref, cen_ref, dist_ref):
    x = pc_ref[0]
    y = pc_ref[1]
    z = pc_ref[2]
    col = lax.broadcasted_iota(jnp.int32, (B, N), 1)
    kcol = lax.broadcasted_iota(jnp.int32, (B, NG), 1)
    dist_ref[...] = jnp.full((B, N), 1e10, jnp.float32)

    def step(k, carry):
        far, cxs, cys, czs = carry
        onehot = col == far
        cx = jnp.sum(jnp.where(onehot, x, 0.0), axis=1, keepdims=True)
        cy = jnp.sum(jnp.where(onehot, y, 0.0), axis=1, keepdims=True)
        cz = jnp.sum(jnp.where(onehot, z, 0.0), axis=1, keepdims=True)
        sel = kcol == k
        cxs = jnp.where(sel, cx, cxs)
        cys = jnp.where(sel, cy, cys)
        czs = jnp.where(sel, cz, czs)
        dx = x - cx
        dy = y - cy
        dz = z - cz
        d = dx * dx + dy * dy + dz * dz
        dist = jnp.minimum(dist_ref[...], d)
        dist_ref[...] = dist
        m = jnp.max(dist, axis=1, keepdims=True)
        far = jnp.min(jnp.where(dist == m, col, N), axis=1, keepdims=True)
        return far, cxs, cys, czs

    zero = jnp.zeros((B, NG), jnp.float32)
    _, cxs, cys, czs = lax.fori_loop(
        0, NG, step, (far0_ref[...], zero, zero, zero))
    cen_ref[0] = cxs
    cen_ref[1] = cys
    cen_ref[2] = czs


def _run_fps(pc_t, far0):
    return pl.pallas_call(
        _fps_body,
        out_shape=jax.ShapeDtypeStruct((3, B, NG), jnp.float32),
        scratch_shapes=[pltpu.VMEM((B, N), jnp.float32)],
    )(pc_t, far0)


NB = 64
W = N // NB
T = 8
_BIG = 1 << 30
_INF = float("inf")


def _knn_dist(pc_ref, cen_ref):
    px = pc_ref[0, 0:1, :]
    py = pc_ref[0, 1:2, :]
    pz = pc_ref[0, 2:3, :]
    cx = cen_ref[0, :, 0:1]
    cy = cen_ref[0, :, 1:2]
    cz = cen_ref[0, :, 2:3]
    sq_p = (px * px + py * py) + pz * pz
    sq_c = (cx * cx + cy * cy) + cz * cz
    bpx = px.astype(jnp.bfloat16).astype(jnp.float32)
    bpy = py.astype(jnp.bfloat16).astype(jnp.float32)
    bpz = pz.astype(jnp.bfloat16).astype(jnp.float32)
    bcx = cx.astype(jnp.bfloat16).astype(jnp.float32)
    bcy = cy.astype(jnp.bfloat16).astype(jnp.float32)
    bcz = cz.astype(jnp.bfloat16).astype(jnp.float32)
    dot = bcx * bpx + bcy * bpy + bcz * bpz
    d = (sq_c + sq_p) - 2.0 * dot
    return d.reshape(NG, NB, W)


def _knn_body(pc_ref, cen_ref, idx_ref, d_ref, bm_ref, bi_ref):
    d_ref[...] = _knn_dist(pc_ref, cen_ref)

    lane = lax.broadcasted_iota(jnp.int32, (NG, NB, W), 2)
    jblk = lax.broadcasted_iota(jnp.int32, (NG, NB), 1)
    kcol = lax.broadcasted_iota(jnp.int32, (NG, GS), 1)

    def tpass(t, carry):
        e = d_ref[...]
        m = jnp.min(e, axis=2)
        i = jnp.min(jnp.where(e == m[:, :, None], lane, W), axis=2)
        d_ref[...] = jnp.where(lane == i[:, :, None], _INF, e)
        bm_ref[pl.ds(t, 1)] = m[None]
        bi_ref[pl.ds(t, 1)] = (jblk * W + i)[None]
        return carry

    lax.fori_loop(0, T, tpass, 0)

    cur, cidx = bm_ref[0], bi_ref[0]
    pops = jnp.zeros((NG, NB), jnp.int32)
    acc = jnp.zeros((NG, GS), jnp.int32)

    def step(k, carry):
        cur, cidx, pops, acc = carry
        m = jnp.min(cur, axis=1, keepdims=True)
        isel = jnp.min(jnp.where(cur == m, cidx, _BIG), axis=1, keepdims=True)
        acc = jnp.where(kcol == k, isel, acc)
        popped = cidx == isel
        pops = pops + popped.astype(jnp.int32)
        ncur = jnp.full((NG, NB), _INF)
        ncid = jnp.full((NG, NB), _BIG)
        for t in range(1, T):
            lv = pops == t
            ncur = jnp.where(lv, bm_ref[t], ncur)
            ncid = jnp.where(lv, bi_ref[t], ncid)
        cur = jnp.where(popped, ncur, cur)
        cidx = jnp.where(popped, ncid, cidx)
        return cur, cidx, pops, acc

    _, _, pops, acc = lax.fori_loop(0, GS, step, (cur, cidx, pops, acc))
    overflow = jnp.max(pops) >= T

    def fallback():
        d_ref[...] = _knn_dist(pc_ref, cen_ref)
        gcol = jblk[:, :, None] * W + lane

        def fstep(k, facc):
            d = d_ref[...]
            m = jnp.min(jnp.min(d, axis=2), axis=1, keepdims=True)
            i = jnp.min(jnp.min(
                jnp.where(d == m[:, :, None], gcol, _BIG), axis=2),
                axis=1, keepdims=True)
            d_ref[...] = jnp.where(gcol == i[:, :, None], _INF, d)
            return jnp.where(kcol == k, i, facc)

        return lax.fori_loop(0, GS, fstep, jnp.zeros((NG, GS), jnp.int32))

    idx_ref[0] = lax.cond(overflow, fallback, lambda: acc)


def _run_knn(pc_bt, center):
    return pl.pallas_call(
        _knn_body,
        grid=(B,),
        in_specs=[
            pl.BlockSpec((1, 3, N), lambda b: (b, 0, 0)),
            pl.BlockSpec((1, NG, 3), lambda b: (b, 0, 0)),
        ],
        out_specs=pl.BlockSpec((1, NG, GS), lambda b: (b, 0, 0)),
        out_shape=jax.ShapeDtypeStruct((B, NG, GS), jnp.int32),
        scratch_shapes=[pltpu.VMEM((NG, NB, W), jnp.float32),
                        pltpu.VMEM((T, NG, NB), jnp.float32),
                        pltpu.VMEM((T, NG, NB), jnp.int32)],
    )(pc_bt, center)


_NC, _NS, _L = 2, 16, 16


def _gather_body(pc_hbm, idx_hbm, cen_hbm, out_hbm, xv, yv, zv, iv, cv, ov):
    b = lax.axis_index("s") * _NC + lax.axis_index("c")
    pltpu.sync_copy(pc_hbm.at[pl.ds((3 * b + 0) * N, N)], xv)
    pltpu.sync_copy(pc_hbm.at[pl.ds((3 * b + 1) * N, N)], yv)
    pltpu.sync_copy(pc_hbm.at[pl.ds((3 * b + 2) * N, N)], zv)
    pltpu.sync_copy(idx_hbm.at[pl.ds(b * (NG * GS), NG * GS)], iv)
    pltpu.sync_copy(cen_hbm.at[pl.ds(b * (NG * 3), NG * 3)], cv)
    lane = lax.iota(jnp.int32, _L)

    def group(g, carry):
        c0 = jnp.full((_L,), 3 * g, jnp.int32)
        cxv = plsc.load_gather(cv, [c0])
        cyv = plsc.load_gather(cv, [c0 + 1])
        czv = plsc.load_gather(cv, [c0 + 2])
        base = g * GS
        for h in range(GS // _L):
            off = base + h * _L
            idx = iv[pl.ds(off, _L)]
            gx = plsc.load_gather(xv, [idx]) - cxv
            gy = plsc.load_gather(yv, [idx]) - cyv
            gz = plsc.load_gather(zv, [idx]) - czv
            pos = (off + lane) * 3
            plsc.store_scatter(ov, [pos], gx)
            plsc.store_scatter(ov, [pos + 1], gy)
            plsc.store_scatter(ov, [pos + 2], gz)
        return carry

    lax.fori_loop(0, NG, group, 0)
    pltpu.sync_copy(ov, out_hbm.at[pl.ds(b * (NG * GS * 3), NG * GS * 3)])


def _run_gather(pc_bt, idx_flat, cen_flat):
    mesh = plsc.VectorSubcoreMesh(core_axis_name="c", subcore_axis_name="s")
    f = functools.partial(
        pl.kernel,
        out_type=jax.ShapeDtypeStruct((B * NG * GS * 3,), jnp.float32),
        mesh=mesh,
        scratch_types=[
            pltpu.VMEM((N,), jnp.float32),
            pltpu.VMEM((N,), jnp.float32),
            pltpu.VMEM((N,), jnp.float32),
            pltpu.VMEM((NG * GS,), jnp.int32),
            pltpu.VMEM((NG * 3,), jnp.float32),
            pltpu.VMEM((NG * GS * 3,), jnp.float32),
        ],
        compiler_params=pltpu.CompilerParams(needs_layout_passes=False),
    )(_gather_body)
    return f(pc_bt, idx_flat, cen_flat)


def kernel(pc, key):
    far0 = jax.random.randint(key, (B,), 0, N).astype(jnp.int32).reshape(B, 1)
    pc_t = jnp.transpose(pc, (2, 0, 1))
    pc_bt = jnp.transpose(pc, (0, 2, 1))

    cen_t = _run_fps(pc_t, far0)
    center = jnp.transpose(cen_t, (1, 2, 0))

    idx = _run_knn(pc_bt, center)

    nbh = _run_gather(pc_bt.reshape(-1), idx.reshape(-1), center.reshape(-1))
    neighborhood = nbh.reshape(B, NG, GS, 3)
    return neighborhood, center


# speedup vs baseline: 1.6878x; 1.6878x over previous
import functools

import jax
import jax.numpy as jnp
from jax import lax
from jax.experimental import pallas as pl
from jax.experimental.pallas import tpu as pltpu
from jax.experimental.pallas import tpu_sc as plsc

B = 32
N = 8192
NG = 256
GS = 32


def _fps_body(pc_ref, far0_ref, cen_ref, dist_ref):
    x = pc_ref[0]
    y = pc_ref[1]
    z = pc_ref[2]
    col = lax.broadcasted_iota(jnp.int32, (B, N), 1)
    kcol = lax.broadcasted_iota(jnp.int32, (B, NG), 1)
    dist_ref[...] = jnp.full((B, N), 1e10, jnp.float32)

    def step(k, carry):
        far, cxs, cys, czs = carry
        onehot = col == far
        cx = jnp.sum(jnp.where(onehot, x, 0.0), axis=1, keepdims=True)
        cy = jnp.sum(jnp.where(onehot, y, 0.0), axis=1, keepdims=True)
        cz = jnp.sum(jnp.where(onehot, z, 0.0), axis=1, keepdims=True)
        sel = kcol == k
        cxs = jnp.where(sel, cx, cxs)
        cys = jnp.where(sel, cy, cys)
        czs = jnp.where(sel, cz, czs)
        dx = x - cx
        dy = y - cy
        dz = z - cz
        d = dx * dx + dy * dy + dz * dz
        dist = jnp.minimum(dist_ref[...], d)
        dist_ref[...] = dist
        m = jnp.max(dist, axis=1, keepdims=True)
        far = jnp.min(jnp.where(dist == m, col, N), axis=1, keepdims=True)
        return far, cxs, cys, czs

    zero = jnp.zeros((B, NG), jnp.float32)
    _, cxs, cys, czs = lax.fori_loop(
        0, NG, step, (far0_ref[...], zero, zero, zero))
    cen_ref[0] = cxs
    cen_ref[1] = cys
    cen_ref[2] = czs


def _run_fps(pc_t, far0):
    return pl.pallas_call(
        _fps_body,
        out_shape=jax.ShapeDtypeStruct((3, B, NG), jnp.float32),
        scratch_shapes=[pltpu.VMEM((B, N), jnp.float32)],
    )(pc_t, far0)


NB = 128
W = N // NB
T = 6
_BIG = 1 << 30
_INF = float("inf")


def _knn_dist(pc_ref, cen_ref):
    px = pc_ref[0, 0]
    py = pc_ref[0, 1]
    pz = pc_ref[0, 2]
    cx = cen_ref[0, :, 0:1][None]
    cy = cen_ref[0, :, 1:2][None]
    cz = cen_ref[0, :, 2:3][None]
    sq_p = (px * px + py * py) + pz * pz
    sq_c = (cx * cx + cy * cy) + cz * cz
    bpx = px.astype(jnp.bfloat16).astype(jnp.float32)
    bpy = py.astype(jnp.bfloat16).astype(jnp.float32)
    bpz = pz.astype(jnp.bfloat16).astype(jnp.float32)
    bcx = cx.astype(jnp.bfloat16).astype(jnp.float32)
    bcy = cy.astype(jnp.bfloat16).astype(jnp.float32)
    bcz = cz.astype(jnp.bfloat16).astype(jnp.float32)
    dot = bcx * bpx + bcy * bpy + bcz * bpz
    return (sq_c + sq_p) - 2.0 * dot


def _knn_body(pc_ref, cen_ref, idx_ref, d_ref, bm_ref, bi_ref):
    d_ref[...] = _knn_dist(pc_ref, cen_ref)

    wio = lax.broadcasted_iota(jnp.int32, (W, NG, NB), 0)
    jlane = lax.broadcasted_iota(jnp.int32, (NG, NB), 1)
    kcol = lax.broadcasted_iota(jnp.int32, (NG, GS), 1)

    def tpass(t, carry):
        e = d_ref[...]
        m = jnp.min(e, axis=0)
        i = jnp.min(jnp.where(e == m[None], wio, W), axis=0)
        d_ref[...] = jnp.where(wio == i[None], _INF, e)
        bm_ref[pl.ds(t, 1)] = m[None]
        bi_ref[pl.ds(t, 1)] = (i * NB + jlane)[None]
        return carry

    lax.fori_loop(0, T, tpass, 0)

    cand_v0 = jnp.concatenate([bm_ref[t] for t in range(T)], axis=1)
    cand_i = jnp.concatenate([bi_ref[t] for t in range(T)], axis=1)
    acc = jnp.zeros((NG, GS), jnp.int32)

    def step(k, carry):
        cand_v, acc, _ = carry
        m = jnp.min(cand_v, axis=1, keepdims=True)
        isel = jnp.min(jnp.where(cand_v == m, cand_i, _BIG), axis=1,
                       keepdims=True)
        acc = jnp.where(kcol == k, isel, acc)
        cand_v = jnp.where(cand_i == isel, _INF, cand_v)
        return cand_v, acc, m

    _, acc, vlast = lax.fori_loop(
        0, GS, step, (cand_v0, acc, jnp.zeros((NG, 1), jnp.float32)))

    deep = jnp.min(bm_ref[T - 1], axis=1, keepdims=True)
    overflow = jnp.any(vlast >= deep)

    def fallback():
        d_ref[...] = _knn_dist(pc_ref, cen_ref)
        gidx = wio * NB + jlane[None]

        def fstep(k, facc):
            e = d_ref[...]
            bm = jnp.min(e, axis=0)
            bi = jnp.min(jnp.where(e == bm[None], wio, W), axis=0)
            gi = bi * NB + jlane
            m = jnp.min(bm, axis=1, keepdims=True)
            isel = jnp.min(jnp.where(bm == m, gi, _BIG), axis=1,
                           keepdims=True)
            facc = jnp.where(kcol == k, isel, facc)
            d_ref[...] = jnp.where(gidx == isel[None], _INF, e)
            return facc

        return lax.fori_loop(0, GS, fstep, jnp.zeros((NG, GS), jnp.int32))

    idx_ref[0] = lax.cond(overflow, fallback, lambda: acc)


def _run_knn(pc_knn, center):
    return pl.pallas_call(
        _knn_body,
        grid=(B,),
        in_specs=[
            pl.BlockSpec((1, 3, W, 1, NB), lambda b: (b, 0, 0, 0, 0)),
            pl.BlockSpec((1, NG, 3), lambda b: (b, 0, 0)),
        ],
        out_specs=pl.BlockSpec((1, NG, GS), lambda b: (b, 0, 0)),
        out_shape=jax.ShapeDtypeStruct((B, NG, GS), jnp.int32),
        scratch_shapes=[pltpu.VMEM((W, NG, NB), jnp.float32),
                        pltpu.VMEM((T, NG, NB), jnp.float32),
                        pltpu.VMEM((T, NG, NB), jnp.int32)],
    )(pc_knn, center)


_NC, _NS, _L = 2, 16, 16


def _gather_body(pc_hbm, idx_hbm, cen_hbm, out_hbm, xv, yv, zv, iv, cv, ov):
    b = lax.axis_index("s") * _NC + lax.axis_index("c")
    pltpu.sync_copy(pc_hbm.at[pl.ds((3 * b + 0) * N, N)], xv)
    pltpu.sync_copy(pc_hbm.at[pl.ds((3 * b + 1) * N, N)], yv)
    pltpu.sync_copy(pc_hbm.at[pl.ds((3 * b + 2) * N, N)], zv)
    pltpu.sync_copy(idx_hbm.at[pl.ds(b * (NG * GS), NG * GS)], iv)
    pltpu.sync_copy(cen_hbm.at[pl.ds(b * (NG * 3), NG * 3)], cv)
    lane = lax.iota(jnp.int32, _L)

    def group(g, carry):
        c0 = jnp.full((_L,), 3 * g, jnp.int32)
        cxv = plsc.load_gather(cv, [c0])
        cyv = plsc.load_gather(cv, [c0 + 1])
        czv = plsc.load_gather(cv, [c0 + 2])
        base = g * GS
        for h in range(GS // _L):
            off = base + h * _L
            idx = iv[pl.ds(off, _L)]
            gx = plsc.load_gather(xv, [idx]) - cxv
            gy = plsc.load_gather(yv, [idx]) - cyv
            gz = plsc.load_gather(zv, [idx]) - czv
            pos = (off + lane) * 3
            plsc.store_scatter(ov, [pos], gx)
            plsc.store_scatter(ov, [pos + 1], gy)
            plsc.store_scatter(ov, [pos + 2], gz)
        return carry

    lax.fori_loop(0, NG, group, 0)
    pltpu.sync_copy(ov, out_hbm.at[pl.ds(b * (NG * GS * 3), NG * GS * 3)])


def _run_gather(pc_bt, idx_flat, cen_flat):
    mesh = plsc.VectorSubcoreMesh(core_axis_name="c", subcore_axis_name="s")
    f = functools.partial(
        pl.kernel,
        out_type=jax.ShapeDtypeStruct((B * NG * GS * 3,), jnp.float32),
        mesh=mesh,
        scratch_types=[
            pltpu.VMEM((N,), jnp.float32),
            pltpu.VMEM((N,), jnp.float32),
            pltpu.VMEM((N,), jnp.float32),
            pltpu.VMEM((NG * GS,), jnp.int32),
            pltpu.VMEM((NG * 3,), jnp.float32),
            pltpu.VMEM((NG * GS * 3,), jnp.float32),
        ],
        compiler_params=pltpu.CompilerParams(needs_layout_passes=False),
    )(_gather_body)
    return f(pc_bt, idx_flat, cen_flat)


def kernel(pc, key):
    far0 = jax.random.randint(key, (B,), 0, N).astype(jnp.int32).reshape(B, 1)
    pc_t = jnp.transpose(pc, (2, 0, 1))
    pc_bt = jnp.transpose(pc, (0, 2, 1))

    cen_t = _run_fps(pc_t, far0)
    center = jnp.transpose(cen_t, (1, 2, 0))

    idx = _run_knn(pc_bt.reshape(B, 3, W, 1, NB), center)

    nbh = _run_gather(pc_bt.reshape(-1), idx.reshape(-1), center.reshape(-1))
    neighborhood = nbh.reshape(B, NG, GS, 3)
    return neighborhood, center


# speedup vs baseline: 1.7707x; 1.0491x over previous
import functools

import jax
import jax.numpy as jnp
from jax import lax
from jax.experimental import pallas as pl
from jax.experimental.pallas import tpu as pltpu
from jax.experimental.pallas import tpu_sc as plsc

B = 32
N = 8192
NG = 256
GS = 32


def _fps_body(pc_ref, far0_ref, cen_ref, dist_ref):
    x = pc_ref[0]
    y = pc_ref[1]
    z = pc_ref[2]
    col = lax.broadcasted_iota(jnp.int32, (B, N), 1)
    kcol = lax.broadcasted_iota(jnp.int32, (B, NG), 1)
    dist_ref[...] = jnp.full((B, N), 1e10, jnp.float32)

    def step(k, carry):
        far, cxs, cys, czs = carry
        onehot = col == far
        cx = jnp.sum(jnp.where(onehot, x, 0.0), axis=1, keepdims=True)
        cy = jnp.sum(jnp.where(onehot, y, 0.0), axis=1, keepdims=True)
        cz = jnp.sum(jnp.where(onehot, z, 0.0), axis=1, keepdims=True)
        sel = kcol == k
        cxs = jnp.where(sel, cx, cxs)
        cys = jnp.where(sel, cy, cys)
        czs = jnp.where(sel, cz, czs)
        dx = x - cx
        dy = y - cy
        dz = z - cz
        d = dx * dx + dy * dy + dz * dz
        dist = jnp.minimum(dist_ref[...], d)
        dist_ref[...] = dist
        m = jnp.max(dist, axis=1, keepdims=True)
        far = jnp.min(jnp.where(dist == m, col, N), axis=1, keepdims=True)
        return far, cxs, cys, czs

    zero = jnp.zeros((B, NG), jnp.float32)
    _, cxs, cys, czs = lax.fori_loop(
        0, NG, step, (far0_ref[...], zero, zero, zero))
    cen_ref[0] = cxs
    cen_ref[1] = cys
    cen_ref[2] = czs


def _run_fps(pc_t, far0):
    return pl.pallas_call(
        _fps_body,
        out_shape=jax.ShapeDtypeStruct((3, B, NG), jnp.float32),
        scratch_shapes=[pltpu.VMEM((B, N), jnp.float32)],
    )(pc_t, far0)


NB = 128
W = N // NB
T = 6
_BIG = 1 << 30
_INF = float("inf")


def _knn_dist(pc_ref, cen_ref):
    px = pc_ref[0, 0]
    py = pc_ref[0, 1]
    pz = pc_ref[0, 2]
    cx = cen_ref[0, :, 0:1][None]
    cy = cen_ref[0, :, 1:2][None]
    cz = cen_ref[0, :, 2:3][None]
    sq_p = (px * px + py * py) + pz * pz
    sq_c = (cx * cx + cy * cy) + cz * cz
    bpx = px.astype(jnp.bfloat16).astype(jnp.float32)
    bpy = py.astype(jnp.bfloat16).astype(jnp.float32)
    bpz = pz.astype(jnp.bfloat16).astype(jnp.float32)
    bcx = cx.astype(jnp.bfloat16).astype(jnp.float32)
    bcy = cy.astype(jnp.bfloat16).astype(jnp.float32)
    bcz = cz.astype(jnp.bfloat16).astype(jnp.float32)
    dot = bcx * bpx + bcy * bpy + bcz * bpz
    return (sq_c + sq_p) - 2.0 * dot


def _knn_body(pc_ref, cen_ref, idx_ref, d_ref, bm_ref, bi_ref):
    d_ref[...] = _knn_dist(pc_ref, cen_ref)

    wio = lax.broadcasted_iota(jnp.int32, (W, NG, NB), 0)
    jlane = lax.broadcasted_iota(jnp.int32, (NG, NB), 1)
    kcol = lax.broadcasted_iota(jnp.int32, (NG, GS), 1)

    def tpass(t, carry):
        e = d_ref[...]
        m = jnp.min(e, axis=0)
        i = jnp.min(jnp.where(e == m[None], wio, W), axis=0)
        d_ref[...] = jnp.where(wio == i[None], _INF, e)
        bm_ref[pl.ds(t, 1)] = m[None]
        bi_ref[pl.ds(t, 1)] = (i * NB + jlane)[None]
        return carry

    lax.fori_loop(0, T, tpass, 0)

    acc = jnp.zeros((NG, GS), jnp.int32)

    def step(k, carry):
        cur, cidx, pops, acc, _ = carry
        m = jnp.min(cur, axis=1, keepdims=True)
        isel = jnp.min(jnp.where(cur == m, cidx, _BIG), axis=1,
                       keepdims=True)
        acc = jnp.where(kcol == k, isel, acc)
        popped = cidx == isel
        pops = pops + popped.astype(jnp.int32)
        ncur = jnp.full((NG, NB), _INF)
        ncid = jnp.full((NG, NB), _BIG)
        for t in range(1, T):
            lv = pops == t
            ncur = jnp.where(lv, bm_ref[t], ncur)
            ncid = jnp.where(lv, bi_ref[t], ncid)
        cur = jnp.where(popped, ncur, cur)
        cidx = jnp.where(popped, ncid, cidx)
        return cur, cidx, pops, acc, m

    _, _, _, acc, vlast = lax.fori_loop(
        0, GS, step,
        (bm_ref[0], bi_ref[0], jnp.zeros((NG, NB), jnp.int32), acc,
         jnp.zeros((NG, 1), jnp.float32)))

    deep = jnp.min(bm_ref[T - 1], axis=1, keepdims=True)
    overflow = jnp.any(vlast >= deep)

    def fallback():
        d_ref[...] = _knn_dist(pc_ref, cen_ref)
        gidx = wio * NB + jlane[None]

        def fstep(k, facc):
            e = d_ref[...]
            bm = jnp.min(e, axis=0)
            bi = jnp.min(jnp.where(e == bm[None], wio, W), axis=0)
            gi = bi * NB + jlane
            m = jnp.min(bm, axis=1, keepdims=True)
            isel = jnp.min(jnp.where(bm == m, gi, _BIG), axis=1,
                           keepdims=True)
            facc = jnp.where(kcol == k, isel, facc)
            d_ref[...] = jnp.where(gidx == isel[None], _INF, e)
            return facc

        return lax.fori_loop(0, GS, fstep, jnp.zeros((NG, GS), jnp.int32))

    idx_ref[0] = lax.cond(overflow, fallback, lambda: acc)


def _run_knn(pc_knn, center):
    return pl.pallas_call(
        _knn_body,
        grid=(B,),
        in_specs=[
            pl.BlockSpec((1, 3, W, 1, NB), lambda b: (b, 0, 0, 0, 0)),
            pl.BlockSpec((1, NG, 3), lambda b: (b, 0, 0)),
        ],
        out_specs=pl.BlockSpec((1, NG, GS), lambda b: (b, 0, 0)),
        out_shape=jax.ShapeDtypeStruct((B, NG, GS), jnp.int32),
        scratch_shapes=[pltpu.VMEM((W, NG, NB), jnp.float32),
                        pltpu.VMEM((T, NG, NB), jnp.float32),
                        pltpu.VMEM((T, NG, NB), jnp.int32)],
    )(pc_knn, center)


_NC, _NS, _L = 2, 16, 16


def _gather_body(pc_hbm, idx_hbm, cen_hbm, out_hbm, xv, yv, zv, iv, cv, ov):
    b = lax.axis_index("s") * _NC + lax.axis_index("c")
    pltpu.sync_copy(pc_hbm.at[pl.ds((3 * b + 0) * N, N)], xv)
    pltpu.sync_copy(pc_hbm.at[pl.ds((3 * b + 1) * N, N)], yv)
    pltpu.sync_copy(pc_hbm.at[pl.ds((3 * b + 2) * N, N)], zv)
    pltpu.sync_copy(idx_hbm.at[pl.ds(b * (NG * GS), NG * GS)], iv)
    pltpu.sync_copy(cen_hbm.at[pl.ds(b * (NG * 3), NG * 3)], cv)
    lane = lax.iota(jnp.int32, _L)

    def group(g, carry):
        c0 = jnp.full((_L,), 3 * g, jnp.int32)
        cxv = plsc.load_gather(cv, [c0])
        cyv = plsc.load_gather(cv, [c0 + 1])
        czv = plsc.load_gather(cv, [c0 + 2])
        base = g * GS
        for h in range(GS // _L):
            off = base + h * _L
            idx = iv[pl.ds(off, _L)]
            gx = plsc.load_gather(xv, [idx]) - cxv
            gy = plsc.load_gather(yv, [idx]) - cyv
            gz = plsc.load_gather(zv, [idx]) - czv
            pos = (off + lane) * 3
            plsc.store_scatter(ov, [pos], gx)
            plsc.store_scatter(ov, [pos + 1], gy)
            plsc.store_scatter(ov, [pos + 2], gz)
        return carry

    lax.fori_loop(0, NG, group, 0)
    pltpu.sync_copy(ov, out_hbm.at[pl.ds(b * (NG * GS * 3), NG * GS * 3)])


def _run_gather(pc_bt, idx_flat, cen_flat):
    mesh = plsc.VectorSubcoreMesh(core_axis_name="c", subcore_axis_name="s")
    f = functools.partial(
        pl.kernel,
        out_type=jax.ShapeDtypeStruct((B * NG * GS * 3,), jnp.float32),
        mesh=mesh,
        scratch_types=[
            pltpu.VMEM((N,), jnp.float32),
            pltpu.VMEM((N,), jnp.float32),
            pltpu.VMEM((N,), jnp.float32),
            pltpu.VMEM((NG * GS,), jnp.int32),
            pltpu.VMEM((NG * 3,), jnp.float32),
            pltpu.VMEM((NG * GS * 3,), jnp.float32),
        ],
        compiler_params=pltpu.CompilerParams(needs_layout_passes=False),
    )(_gather_body)
    return f(pc_bt, idx_flat, cen_flat)


def kernel(pc, key):
    far0 = jax.random.randint(key, (B,), 0, N).astype(jnp.int32).reshape(B, 1)
    pc_t = jnp.transpose(pc, (2, 0, 1))
    pc_bt = jnp.transpose(pc, (0, 2, 1))

    cen_t = _run_fps(pc_t, far0)
    center = jnp.transpose(cen_t, (1, 2, 0))

    idx = _run_knn(pc_bt.reshape(B, 3, W, 1, NB), center)

    nbh = _run_gather(pc_bt.reshape(-1), idx.reshape(-1), center.reshape(-1))
    neighborhood = nbh.reshape(B, NG, GS, 3)
    return neighborhood, center


# speedup vs baseline: 2.0101x; 1.1352x over previous
import functools

import jax
import jax.numpy as jnp
from jax import lax
from jax.experimental import pallas as pl
from jax.experimental.pallas import tpu as pltpu
from jax.experimental.pallas import tpu_sc as plsc

B = 32
N = 8192
NG = 256
GS = 32


def _fps_body(pc_ref, far0_ref, cen_ref, dist_ref):
    x = pc_ref[0]
    y = pc_ref[1]
    z = pc_ref[2]
    col = lax.broadcasted_iota(jnp.int32, (B, N), 1)
    kcol = lax.broadcasted_iota(jnp.int32, (B, NG), 1)
    dist_ref[...] = jnp.full((B, N), 1e10, jnp.float32)

    def step(k, carry):
        far, cxs, cys, czs = carry
        onehot = col == far
        cx = jnp.sum(jnp.where(onehot, x, 0.0), axis=1, keepdims=True)
        cy = jnp.sum(jnp.where(onehot, y, 0.0), axis=1, keepdims=True)
        cz = jnp.sum(jnp.where(onehot, z, 0.0), axis=1, keepdims=True)
        sel = kcol == k
        cxs = jnp.where(sel, cx, cxs)
        cys = jnp.where(sel, cy, cys)
        czs = jnp.where(sel, cz, czs)
        dx = x - cx
        dy = y - cy
        dz = z - cz
        d = dx * dx + dy * dy + dz * dz
        dist = jnp.minimum(dist_ref[...], d)
        dist_ref[...] = dist
        m = jnp.max(dist, axis=1, keepdims=True)
        far = jnp.min(jnp.where(dist == m, col, N), axis=1, keepdims=True)
        return far, cxs, cys, czs

    zero = jnp.zeros((B, NG), jnp.float32)
    _, cxs, cys, czs = lax.fori_loop(
        0, NG, step, (far0_ref[...], zero, zero, zero))
    cen_ref[0] = cxs
    cen_ref[1] = cys
    cen_ref[2] = czs


def _run_fps(pc_t, far0):
    return pl.pallas_call(
        _fps_body,
        out_shape=jax.ShapeDtypeStruct((3, B, NG), jnp.float32),
        scratch_shapes=[pltpu.VMEM((B, N), jnp.float32)],
    )(pc_t, far0)


NB = 128
W = N // NB
T = 6
_BIG = 1 << 30
_INF = float("inf")


def _knn_dist(pc_ref, cen_ref):
    px = pc_ref[0, 0]
    py = pc_ref[0, 1]
    pz = pc_ref[0, 2]
    cx = cen_ref[0, :, 0:1][None]
    cy = cen_ref[0, :, 1:2][None]
    cz = cen_ref[0, :, 2:3][None]
    sq_p = (px * px + py * py) + pz * pz
    sq_c = (cx * cx + cy * cy) + cz * cz
    bpx = px.astype(jnp.bfloat16).astype(jnp.float32)
    bpy = py.astype(jnp.bfloat16).astype(jnp.float32)
    bpz = pz.astype(jnp.bfloat16).astype(jnp.float32)
    bcx = cx.astype(jnp.bfloat16).astype(jnp.float32)
    bcy = cy.astype(jnp.bfloat16).astype(jnp.float32)
    bcz = cz.astype(jnp.bfloat16).astype(jnp.float32)
    dot = bcx * bpx + bcy * bpy + bcz * bpz
    return (sq_c + sq_p) - 2.0 * dot


def _knn_body(pc_ref, cen_ref, idx_ref, d_ref, bm_ref, bi_ref):
    cx = cen_ref[0, :, 0:1]
    cy = cen_ref[0, :, 1:2]
    cz = cen_ref[0, :, 2:3]
    sq_c = (cx * cx + cy * cy) + cz * cz
    bcx = cx.astype(jnp.bfloat16).astype(jnp.float32)
    bcy = cy.astype(jnp.bfloat16).astype(jnp.float32)
    bcz = cz.astype(jnp.bfloat16).astype(jnp.float32)

    wio = lax.broadcasted_iota(jnp.int32, (W, NG, NB), 0)
    jlane = lax.broadcasted_iota(jnp.int32, (NG, NB), 1)
    kcol = lax.broadcasted_iota(jnp.int32, (NG, GS), 1)

    m = None
    i = None
    for w in range(W):
        px = pc_ref[0, 0, w]
        py = pc_ref[0, 1, w]
        pz = pc_ref[0, 2, w]
        sq_p = (px * px + py * py) + pz * pz
        bpx = px.astype(jnp.bfloat16).astype(jnp.float32)
        bpy = py.astype(jnp.bfloat16).astype(jnp.float32)
        bpz = pz.astype(jnp.bfloat16).astype(jnp.float32)
        dot = bcx * bpx + bcy * bpy + bcz * bpz
        dslab = (sq_c + sq_p) - 2.0 * dot
        d_ref[w] = dslab
        if w == 0:
            m, i = dslab, jnp.zeros((NG, NB), jnp.int32)
        else:
            cond = dslab < m
            m = jnp.where(cond, dslab, m)
            i = jnp.where(cond, w, i)
    bm_ref[0] = m
    bi_ref[0] = i * NB + jlane

    for t in range(1, T):
        prev_i = i
        m = None
        i = None
        for w in range(W):
            ew = jnp.where(prev_i == w, _INF, d_ref[w])
            if t < T - 1:
                d_ref[w] = ew
            if w == 0:
                m, i = ew, jnp.zeros((NG, NB), jnp.int32)
            else:
                cond = ew < m
                m = jnp.where(cond, ew, m)
                i = jnp.where(cond, w, i)
        bm_ref[t] = m
        bi_ref[t] = i * NB + jlane

    acc = jnp.zeros((NG, GS), jnp.int32)

    def step(k, carry):
        cur, cidx, pops, acc, _ = carry
        m = jnp.min(cur, axis=1, keepdims=True)
        isel = jnp.min(jnp.where(cur == m, cidx, _BIG), axis=1,
                       keepdims=True)
        acc = jnp.where(kcol == k, isel, acc)
        popped = cidx == isel
        pops = pops + popped.astype(jnp.int32)
        ncur = jnp.full((NG, NB), _INF)
        ncid = jnp.full((NG, NB), _BIG)
        for t in range(1, T):
            lv = pops == t
            ncur = jnp.where(lv, bm_ref[t], ncur)
            ncid = jnp.where(lv, bi_ref[t], ncid)
        cur = jnp.where(popped, ncur, cur)
        cidx = jnp.where(popped, ncid, cidx)
        return cur, cidx, pops, acc, m

    _, _, _, acc, vlast = lax.fori_loop(
        0, GS, step,
        (bm_ref[0], bi_ref[0], jnp.zeros((NG, NB), jnp.int32), acc,
         jnp.zeros((NG, 1), jnp.float32)))

    deep = jnp.min(bm_ref[T - 1], axis=1, keepdims=True)
    overflow = jnp.any(vlast >= deep)

    def fallback():
        d_ref[...] = _knn_dist(pc_ref, cen_ref)
        gidx = wio * NB + jlane[None]

        def fstep(k, facc):
            e = d_ref[...]
            bm = jnp.min(e, axis=0)
            bi = jnp.min(jnp.where(e == bm[None], wio, W), axis=0)
            gi = bi * NB + jlane
            m = jnp.min(bm, axis=1, keepdims=True)
            isel = jnp.min(jnp.where(bm == m, gi, _BIG), axis=1,
                           keepdims=True)
            facc = jnp.where(kcol == k, isel, facc)
            d_ref[...] = jnp.where(gidx == isel[None], _INF, e)
            return facc

        return lax.fori_loop(0, GS, fstep, jnp.zeros((NG, GS), jnp.int32))

    idx_ref[0] = lax.cond(overflow, fallback, lambda: acc)


def _run_knn(pc_knn, center):
    return pl.pallas_call(
        _knn_body,
        grid=(B,),
        in_specs=[
            pl.BlockSpec((1, 3, W, 1, NB), lambda b: (b, 0, 0, 0, 0)),
            pl.BlockSpec((1, NG, 3), lambda b: (b, 0, 0)),
        ],
        out_specs=pl.BlockSpec((1, NG, GS), lambda b: (b, 0, 0)),
        out_shape=jax.ShapeDtypeStruct((B, NG, GS), jnp.int32),
        scratch_shapes=[pltpu.VMEM((W, NG, NB), jnp.float32),
                        pltpu.VMEM((T, NG, NB), jnp.float32),
                        pltpu.VMEM((T, NG, NB), jnp.int32)],
    )(pc_knn, center)


_NC, _NS, _L = 2, 16, 16


def _gather_body(pc_hbm, idx_hbm, cen_hbm, out_hbm, xv, yv, zv, iv, cv, ov):
    b = lax.axis_index("s") * _NC + lax.axis_index("c")
    pltpu.sync_copy(pc_hbm.at[pl.ds((3 * b + 0) * N, N)], xv)
    pltpu.sync_copy(pc_hbm.at[pl.ds((3 * b + 1) * N, N)], yv)
    pltpu.sync_copy(pc_hbm.at[pl.ds((3 * b + 2) * N, N)], zv)
    pltpu.sync_copy(idx_hbm.at[pl.ds(b * (NG * GS), NG * GS)], iv)
    pltpu.sync_copy(cen_hbm.at[pl.ds(b * (NG * 3), NG * 3)], cv)
    lane = lax.iota(jnp.int32, _L)

    def group(g, carry):
        c0 = jnp.full((_L,), 3 * g, jnp.int32)
        cxv = plsc.load_gather(cv, [c0])
        cyv = plsc.load_gather(cv, [c0 + 1])
        czv = plsc.load_gather(cv, [c0 + 2])
        base = g * GS
        for h in range(GS // _L):
            off = base + h * _L
            idx = iv[pl.ds(off, _L)]
            gx = plsc.load_gather(xv, [idx]) - cxv
            gy = plsc.load_gather(yv, [idx]) - cyv
            gz = plsc.load_gather(zv, [idx]) - czv
            pos = (off + lane) * 3
            plsc.store_scatter(ov, [pos], gx)
            plsc.store_scatter(ov, [pos + 1], gy)
            plsc.store_scatter(ov, [pos + 2], gz)
        return carry

    lax.fori_loop(0, NG, group, 0)
    pltpu.sync_copy(ov, out_hbm.at[pl.ds(b * (NG * GS * 3), NG * GS * 3)])


def _run_gather(pc_bt, idx_flat, cen_flat):
    mesh = plsc.VectorSubcoreMesh(core_axis_name="c", subcore_axis_name="s")
    f = functools.partial(
        pl.kernel,
        out_type=jax.ShapeDtypeStruct((B * NG * GS * 3,), jnp.float32),
        mesh=mesh,
        scratch_types=[
            pltpu.VMEM((N,), jnp.float32),
            pltpu.VMEM((N,), jnp.float32),
            pltpu.VMEM((N,), jnp.float32),
            pltpu.VMEM((NG * GS,), jnp.int32),
            pltpu.VMEM((NG * 3,), jnp.float32),
            pltpu.VMEM((NG * GS * 3,), jnp.float32),
        ],
        compiler_params=pltpu.CompilerParams(needs_layout_passes=False),
    )(_gather_body)
    return f(pc_bt, idx_flat, cen_flat)


def kernel(pc, key):
    far0 = jax.random.randint(key, (B,), 0, N).astype(jnp.int32).reshape(B, 1)
    pc_t = jnp.transpose(pc, (2, 0, 1))
    pc_bt = jnp.transpose(pc, (0, 2, 1))

    cen_t = _run_fps(pc_t, far0)
    center = jnp.transpose(cen_t, (1, 2, 0))

    idx = _run_knn(pc_bt.reshape(B, 3, W, 1, NB), center)

    nbh = _run_gather(pc_bt.reshape(-1), idx.reshape(-1), center.reshape(-1))
    neighborhood = nbh.reshape(B, NG, GS, 3)
    return neighborhood, center
